# baseline (speedup 1.0000x reference)
import functools

import jax
import jax.numpy as jnp
from jax import lax
from jax.experimental import pallas as pl
from jax.experimental.pallas import tpu as pltpu
from jax.experimental.pallas import tpu_sc as plsc

NC = 2
NS = 16
NW = NC * NS
LANES = 16


def _tc_body(x_ref, wm_ref, m_ref, bsum_ref, msum_ref):
    i = pl.program_id(0)
    wm = wm_ref[...]
    x0 = x_ref[0]
    mx = x0
    for k in (1, 2, 3, 4):
        mx = jnp.maximum(mx, x_ref[k])
    t = jnp.exp(x0 - mx)
    for k in (1, 2, 3, 4):
        t = t + jnp.exp(x_ref[k] - mx)
    lse = jnp.log(t) + mx
    bpart = jnp.sum(wm * lse)
    mpart = jnp.sum(m_ref[...].astype(jnp.float32))

    @pl.when(i == 0)
    def _():
        bsum_ref[0, 0] = 0.0
        msum_ref[0, 0] = 0.0

    bsum_ref[0, 0] += bpart
    msum_ref[0, 0] += mpart


def _tc_call(x_t, wm, m):
    C, B, S = x_t.shape
    bb = 512
    grid = (B // bb,)
    bsum, msum = pl.pallas_call(
        _tc_body,
        grid=grid,
        in_specs=[
            pl.BlockSpec((C, bb, S), lambda i: (0, i, 0)),
            pl.BlockSpec((bb, S), lambda i: (i, 0)),
            pl.BlockSpec((bb, S), lambda i: (i, 0)),
        ],
        out_specs=[
            pl.BlockSpec((1, 1), lambda i: (0, 0), memory_space=pltpu.SMEM),
            pl.BlockSpec((1, 1), lambda i: (0, 0), memory_space=pltpu.SMEM),
        ],
        out_shape=[
            jax.ShapeDtypeStruct((1, 1), jnp.float32),
            jax.ShapeDtypeStruct((1, 1), jnp.float32),
        ],
        compiler_params=pltpu.CompilerParams(
            dimension_semantics=("arbitrary",),
        ),
    )(x_t, wm, m)
    return bsum, msum


def _sc_call(conf_lin, x_lin, wm, idx):
    V5, S = conf_lin.shape
    V = V5 // 5
    B = wm.shape[0]
    NB = B // NW
    CH = 16
    NCH = NB // CH
    GC = 5 * CH
    KS = S // LANES

    mesh = plsc.VectorSubcoreMesh(core_axis_name="c", subcore_axis_name="s")

    @functools.partial(
        pl.kernel,
        out_type=jax.ShapeDtypeStruct((NW * LANES,), jnp.float32),
        mesh=mesh,
        scratch_types=[
            pltpu.VMEM((NB,), jnp.int32),
            pltpu.VMEM((NCH * GC,), jnp.int32),
            pltpu.VMEM((2, GC, S), jnp.float32),
            pltpu.VMEM((2, GC, S), jnp.float32),
            pltpu.VMEM((2, CH, S), jnp.float32),
            pltpu.VMEM((LANES,), jnp.float32),
            pltpu.SemaphoreType.DMA,
            pltpu.SemaphoreType.DMA,
        ],
    )
    def sc_kernel(conf_hbm, x_hbm, wm_hbm, idx_hbm, out_hbm,
                  idx_v, gidx, conf_buf, x_buf, wm_buf, acc_buf,
                  sem0, sem1):
        wid = lax.axis_index("s") * NC + lax.axis_index("c")
        base = wid * NB
        ih = pltpu.async_copy(idx_hbm.at[pl.ds(base, NB)], idx_v, sem0)

        sems = (sem0, sem1)

        def fire_x(t, slot):
            hs = tuple(
                pltpu.async_copy(
                    x_hbm.at[pl.ds(c * B + base + t * CH, CH)],
                    x_buf.at[slot].at[pl.ds(c * CH, CH)], sems[slot])
                for c in range(5))
            h3 = pltpu.async_copy(
                wm_hbm.at[pl.ds(base + t * CH, CH)], wm_buf.at[slot], sems[slot])
            return hs + (h3,)

        def fire_c(t, slot):
            return (pltpu.async_copy(
                conf_hbm.at[gidx.at[pl.ds(t * GC, GC)]], conf_buf.at[slot],
                sems[slot]),)

        p1 = fire_x(0, 0)
        ih.wait()

        def gidx_body(t, carry):
            iv = idx_v[pl.ds(t * CH, CH)]
            for c in range(5):
                gidx[pl.ds(t * GC + c * CH, CH)] = iv + c * V
            return carry

        lax.fori_loop(0, NCH, gidx_body, 0)

        pending = {0: fire_c(0, 0) + p1}
        acc = jnp.zeros((LANES,), jnp.float32)
        for t in range(NCH):
            slot = t % 2
            if t + 1 < NCH:
                pending[t + 1] = fire_c(t + 1, 1 - slot) + fire_x(t + 1, 1 - slot)
            for h in pending.pop(t):
                h.wait()

            def sample_body(i, a):
                def k_body(k, a2):
                    wmv = wm_buf[slot, i, pl.ds(k * LANES, LANES)]
                    for c in range(5):
                        r = c * CH + i
                        a2 = a2 + (conf_buf[slot, r, pl.ds(k * LANES, LANES)]
                                   * x_buf[slot, r, pl.ds(k * LANES, LANES)]
                                   * wmv)
                    return a2
                return lax.fori_loop(0, KS, k_body, a)

            acc = lax.fori_loop(0, CH, sample_body, acc)
        acc_buf[...] = acc
        pltpu.sync_copy(acc_buf, out_hbm.at[pl.ds(wid * LANES, LANES)])

    return sc_kernel(conf_lin, x_lin, wm, idx)


def kernel(outputs, index, pad_mask, weights, confidence):
    B, S, C = outputs.shape
    V = confidence.shape[0]

    x_t = jnp.transpose(outputs, (2, 0, 1))
    conf_t = jnp.transpose(confidence, (2, 0, 1))
    x_lin = x_t.reshape(C * B, S)
    conf_lin = conf_t.reshape(C * V, S)
    wm = weights * pad_mask
    idx = index.astype(jnp.int32)

    bsum, msum = _tc_call(x_t, wm, pad_mask)
    partials = _sc_call(conf_lin, x_lin, wm, idx)
    a = jnp.sum(partials, dtype=jnp.float32)
    return (bsum[0, 0] - a) / msum[0, 0]


# speedup vs baseline: 1.0340x; 1.0340x over previous
import functools

import jax
import jax.numpy as jnp
from jax import lax
from jax.experimental import pallas as pl
from jax.experimental.pallas import tpu as pltpu
from jax.experimental.pallas import tpu_sc as plsc

NC = 2
NS = 16
NW = NC * NS
LANES = 16


def _tc_body(x_ref, wm_ref, m_ref, bsum_ref, msum_ref):
    i = pl.program_id(0)
    wm = wm_ref[...]
    x0 = x_ref[0]
    mx = x0
    for k in (1, 2, 3, 4):
        mx = jnp.maximum(mx, x_ref[k])
    t = jnp.exp(x0 - mx)
    for k in (1, 2, 3, 4):
        t = t + jnp.exp(x_ref[k] - mx)
    lse = jnp.log(t) + mx
    bpart = jnp.sum(wm * lse)
    mpart = jnp.sum(m_ref[...].astype(jnp.float32))

    @pl.when(i == 0)
    def _():
        bsum_ref[0, 0] = 0.0
        msum_ref[0, 0] = 0.0

    bsum_ref[0, 0] += bpart
    msum_ref[0, 0] += mpart


def _tc_call(x_t, wm, m):
    C, B, S = x_t.shape
    bb = 512
    grid = (B // bb,)
    bsum, msum = pl.pallas_call(
        _tc_body,
        grid=grid,
        in_specs=[
            pl.BlockSpec((C, bb, S), lambda i: (0, i, 0)),
            pl.BlockSpec((bb, S), lambda i: (i, 0)),
            pl.BlockSpec((bb, S), lambda i: (i, 0)),
        ],
        out_specs=[
            pl.BlockSpec((1, 1), lambda i: (0, 0), memory_space=pltpu.SMEM),
            pl.BlockSpec((1, 1), lambda i: (0, 0), memory_space=pltpu.SMEM),
        ],
        out_shape=[
            jax.ShapeDtypeStruct((1, 1), jnp.float32),
            jax.ShapeDtypeStruct((1, 1), jnp.float32),
        ],
        compiler_params=pltpu.CompilerParams(
            dimension_semantics=("arbitrary",),
        ),
    )(x_t, wm, m)
    return bsum, msum


def _sc_call(conf_lin, x_lin, wm, idx):
    V5, S = conf_lin.shape
    V = V5 // 5
    B = wm.shape[0]
    NB = B // NW
    CH = 16
    NCH = NB // CH
    GC = 5 * CH
    KS = S // LANES

    mesh = plsc.VectorSubcoreMesh(core_axis_name="c", subcore_axis_name="s")

    @functools.partial(
        pl.kernel,
        out_type=jax.ShapeDtypeStruct((NW * LANES,), jnp.float32),
        mesh=mesh,
        scratch_types=[
            pltpu.VMEM((NB,), jnp.int32),
            pltpu.VMEM((NCH * GC,), jnp.int32),
            pltpu.VMEM((NCH * GC,), jnp.int32),
            pltpu.VMEM((3, GC, S), jnp.float32),
            pltpu.VMEM((3, GC, S), jnp.float32),
            pltpu.VMEM((3, CH, S), jnp.float32),
            pltpu.VMEM((LANES,), jnp.float32),
            pltpu.SemaphoreType.DMA,
            pltpu.SemaphoreType.DMA,
            pltpu.SemaphoreType.DMA,
        ],
    )
    def sc_kernel(conf_hbm, x_hbm, wm_hbm, idx_hbm, out_hbm,
                  idx_v, gidx, xidx, conf_buf, x_buf, wm_buf, acc_buf,
                  sem0, sem1, sem2):
        wid = lax.axis_index("s") * NC + lax.axis_index("c")
        base = wid * NB
        ih = pltpu.async_copy(idx_hbm.at[pl.ds(base, NB)], idx_v, sem0)
        lanes = lax.iota(jnp.int32, LANES)

        def xidx_body(t, carry):
            for c in range(5):
                xidx[pl.ds(t * GC + c * CH, CH)] = (c * B + base) + t * CH + lanes
            return carry

        lax.fori_loop(0, NCH, xidx_body, 0)

        sems = (sem0, sem1, sem2)

        def fire_x(t, slot):
            h2 = pltpu.async_copy(
                x_hbm.at[xidx.at[pl.ds(t * GC, GC)]], x_buf.at[slot], sems[slot])
            h3 = pltpu.async_copy(
                wm_hbm.at[pl.ds(base + t * CH, CH)], wm_buf.at[slot], sems[slot])
            return (h2, h3)

        def fire_c(t, slot):
            return (pltpu.async_copy(
                conf_hbm.at[gidx.at[pl.ds(t * GC, GC)]], conf_buf.at[slot],
                sems[slot]),)

        p1 = fire_x(0, 0)
        ih.wait()

        def gidx_body(t, carry):
            iv = idx_v[pl.ds(t * CH, CH)]
            for c in range(5):
                gidx[pl.ds(t * GC + c * CH, CH)] = iv + c * V
            return carry

        lax.fori_loop(0, NCH, gidx_body, 0)

        pending = {0: fire_c(0, 0) + p1,
                   1: fire_c(1, 1) + fire_x(1, 1)}
        acc = jnp.zeros((LANES,), jnp.float32)
        for t in range(NCH):
            slot = t % 3
            if t + 2 < NCH:
                pending[t + 2] = (fire_c(t + 2, (t + 2) % 3)
                                  + fire_x(t + 2, (t + 2) % 3))
            for h in pending.pop(t):
                h.wait()

            def sample_body(i, a):
                def k_body(k, a2):
                    wmv = wm_buf[slot, i, pl.ds(k * LANES, LANES)]
                    for c in range(5):
                        r = c * CH + i
                        a2 = a2 + (conf_buf[slot, r, pl.ds(k * LANES, LANES)]
                                   * x_buf[slot, r, pl.ds(k * LANES, LANES)]
                                   * wmv)
                    return a2
                return lax.fori_loop(0, KS, k_body, a)

            acc = lax.fori_loop(0, CH, sample_body, acc)
        acc_buf[...] = acc
        pltpu.sync_copy(acc_buf, out_hbm.at[pl.ds(wid * LANES, LANES)])

    return sc_kernel(conf_lin, x_lin, wm, idx)


def kernel(outputs, index, pad_mask, weights, confidence):
    B, S, C = outputs.shape
    V = confidence.shape[0]

    x_t = jnp.transpose(outputs, (2, 0, 1))
    conf_t = jnp.transpose(confidence, (2, 0, 1))
    x_lin = x_t.reshape(C * B, S)
    conf_lin = conf_t.reshape(C * V, S)
    wm = weights * pad_mask
    idx = index.astype(jnp.int32)

    bsum, msum = _tc_call(x_t, wm, pad_mask)
    partials = _sc_call(conf_lin, x_lin, wm, idx)
    a = jnp.sum(partials, dtype=jnp.float32)
    return (bsum[0, 0] - a) / msum[0, 0]
